# baseline (speedup 1.0000x reference)
import functools

import jax
import jax.numpy as jnp
from jax import lax
from jax.experimental import pallas as pl
from jax.experimental.pallas import tpu as pltpu
from jax.experimental.pallas import tpu_sc as plsc

NUM_CODEBOOKS = 32
CODEBOOK_VOCAB_SIZE = 2051
HIDDEN = 2048
BATCH = 4
SEQ = 2048

_INFO = plsc.get_sparse_core_info()
_NC = _INFO.num_cores
_NS = _INFO.num_subcores
NW = _NC * _NS
TOTAL = BATCH * SEQ
BPW = TOTAL // NW
CHUNK = 16
NCHUNK = BPW // CHUNK


NBUF = 2
CHUNKS = [(i * 24, 24) for i in range(10)] + [(240, 16)]
MAXROWS = max(sz for _, sz in CHUNKS)


def _body(idscb_hbm, table_hbm, out_hbm, stg_v, idx_v, rows_v, *sems):
    wid = lax.axis_index("s") * _NC + lax.axis_index("c")
    base = wid * BPW

    pltpu.sync_copy(idscb_hbm.at[wid], stg_v)

    for c in range(NCHUNK):
        idx_v[pl.ds(c * CHUNK, CHUNK)] = (
            stg_v[c, :] + stg_v[NCHUNK + c, :] * CODEBOOK_VOCAB_SIZE
        )

    g_sems, o_sems = sems[:NBUF], sems[NBUF:]
    g_cps = [None] * NBUF
    o_cps = [None] * NBUF

    def fire_gather(k):
        b = k % NBUF
        off, sz = CHUNKS[k]
        g_cps[b] = pltpu.async_copy(
            table_hbm.at[idx_v.at[pl.ds(off, sz)]],
            rows_v.at[b].at[pl.ds(0, sz)],
            g_sems[b],
        )

    def fire_out(k):
        b = k % NBUF
        off, sz = CHUNKS[k]
        o_cps[b] = pltpu.async_copy(
            rows_v.at[b].at[pl.ds(0, sz)],
            out_hbm.at[pl.ds(base + off, sz)],
            o_sems[b],
        )

    nk = len(CHUNKS)
    for k in range(NBUF):
        fire_gather(k)
    for k in range(nk):
        b = k % NBUF
        g_cps[b].wait()
        fire_out(k)
        if k + NBUF < nk:
            o_cps[b].wait()
            fire_gather(k + NBUF)
    for k in range(nk - NBUF, nk):
        o_cps[k % NBUF].wait()


@jax.jit
def kernel(input_ids, codebook_idxs, embed_audio_tokens_weight):
    ids3 = input_ids.astype(jnp.int32).reshape(NW, 1, NCHUNK, CHUNK)
    cb3 = codebook_idxs.astype(jnp.int32).reshape(NW, 1, NCHUNK, CHUNK)
    idscb = jnp.concatenate([ids3, cb3], axis=1).reshape(NW, 2 * NCHUNK, CHUNK)

    mesh = plsc.VectorSubcoreMesh(core_axis_name="c", subcore_axis_name="s")
    run = functools.partial(
        pl.kernel,
        mesh=mesh,
        out_type=jax.ShapeDtypeStruct((TOTAL, HIDDEN), jnp.float32),
        scratch_types=[
            pltpu.VMEM((2 * NCHUNK, CHUNK), jnp.int32),
            pltpu.VMEM((BPW,), jnp.int32),
            pltpu.VMEM((NBUF, MAXROWS, HIDDEN), jnp.float32),
        ] + [pltpu.SemaphoreType.DMA] * (2 * NBUF),
    )(_body)
    out = run(idscb, embed_audio_tokens_weight)
    return out.reshape(BATCH, SEQ, HIDDEN)


# speedup vs baseline: 1.0098x; 1.0098x over previous
import functools

import jax
import jax.numpy as jnp
from jax import lax
from jax.experimental import pallas as pl
from jax.experimental.pallas import tpu as pltpu
from jax.experimental.pallas import tpu_sc as plsc

NUM_CODEBOOKS = 32
CODEBOOK_VOCAB_SIZE = 2051
HIDDEN = 2048
BATCH = 4
SEQ = 2048

_INFO = plsc.get_sparse_core_info()
_NC = _INFO.num_cores
_NS = _INFO.num_subcores
NW = _NC * _NS
TOTAL = BATCH * SEQ
BPW = TOTAL // NW
CHUNK = 16
NCHUNK = BPW // CHUNK


NBUF = 3
HEADC = 4


def _body(idscb_hbm, table_hbm, out_hbm, stg_v, idx_v, rows_v, *sems):
    wid = lax.axis_index("s") * _NC + lax.axis_index("c")
    base = wid * BPW

    nhead = 2 * HEADC
    cp_head = pltpu.async_copy(
        idscb_hbm.at[wid].at[pl.ds(0, nhead)], stg_v.at[pl.ds(0, nhead)],
        sems[2 * NBUF],
    )
    cp_tail = pltpu.async_copy(
        idscb_hbm.at[wid].at[pl.ds(nhead, 2 * NCHUNK - nhead)],
        stg_v.at[pl.ds(nhead, 2 * NCHUNK - nhead)],
        sems[2 * NBUF + 1],
    )

    ntail = NCHUNK - HEADC
    cp_head.wait()
    for c in range(HEADC):
        idx_v[c, :] = stg_v[c, :] + stg_v[HEADC + c, :] * CODEBOOK_VOCAB_SIZE

    g_sems, o_sems = sems[:NBUF], sems[NBUF:2 * NBUF]
    g_cps = [None] * NBUF
    o_cps = [None] * NBUF

    def fire_gather(c):
        b = c % NBUF
        g_cps[b] = pltpu.async_copy(
            table_hbm.at[idx_v.at[c]], rows_v.at[b], g_sems[b]
        )

    def fire_out(c):
        b = c % NBUF
        o_cps[b] = pltpu.async_copy(
            rows_v.at[b], out_hbm.at[pl.ds(base + c * CHUNK, CHUNK)], o_sems[b]
        )

    for c in range(NBUF):
        fire_gather(c)
    cp_tail.wait()
    for c in range(HEADC, NCHUNK):
        idx_v[c, :] = (
            stg_v[nhead + (c - HEADC), :]
            + stg_v[nhead + ntail + (c - HEADC), :] * CODEBOOK_VOCAB_SIZE
        )
    for c in range(NCHUNK):
        b = c % NBUF
        g_cps[b].wait()
        fire_out(c)
        if c + NBUF < NCHUNK:
            o_cps[b].wait()
            fire_gather(c + NBUF)
    for c in range(NCHUNK - NBUF, NCHUNK):
        o_cps[c % NBUF].wait()


@jax.jit
def kernel(input_ids, codebook_idxs, embed_audio_tokens_weight):
    ids3 = input_ids.astype(jnp.int32).reshape(NW, NCHUNK, CHUNK)
    cb3 = codebook_idxs.astype(jnp.int32).reshape(NW, NCHUNK, CHUNK)
    idscb = jnp.concatenate(
        [ids3[:, :HEADC], cb3[:, :HEADC], ids3[:, HEADC:], cb3[:, HEADC:]], axis=1
    )

    mesh = plsc.VectorSubcoreMesh(core_axis_name="c", subcore_axis_name="s")
    run = functools.partial(
        pl.kernel,
        mesh=mesh,
        out_type=jax.ShapeDtypeStruct((TOTAL, HIDDEN), jnp.float32),
        scratch_types=[
            pltpu.VMEM((2 * NCHUNK, CHUNK), jnp.int32),
            pltpu.VMEM((NCHUNK, CHUNK), jnp.int32),
            pltpu.VMEM((NBUF, CHUNK, HIDDEN), jnp.float32),
        ] + [pltpu.SemaphoreType.DMA] * (2 * NBUF + 2),
    )(_body)
    out = run(idscb, embed_audio_tokens_weight)
    return out.reshape(BATCH, SEQ, HIDDEN)
